# baseline (speedup 1.0000x reference)
import functools

import jax
import jax.numpy as jnp
from jax import lax
from jax.experimental import pallas as pl
from jax.experimental.pallas import tpu as pltpu
from jax.experimental.pallas import tpu_sc as plsc

SEQ = 16384
DIM = 32
NC = 2
NS = 16
NW = NC * NS
TPW = SEQ // NW
CHUNK = 128
NCHUNK = TPW // CHUNK
NBLK = TPW // 16


def _sc_body(seq_hbm, emb_hbm, sgn_hbm, out_hbm,
             idx_v, gat_v, sgn_v, tmp_v, part_v, sem, ssem):
    wid = lax.axis_index("s") * NC + lax.axis_index("c")
    base = wid * TPW

    for j in range(NCHUNK):
        pltpu.sync_copy(seq_hbm.at[pl.ds(base + j * CHUNK, CHUNK)], idx_v.at[j])

    sgn_copies = [
        pltpu.make_async_copy(sgn_hbm.at[idx_v.at[j]],
                              sgn_v.at[pl.ds(j * CHUNK, CHUNK)], ssem)
        for j in range(NCHUNK)
    ]
    for c in sgn_copies:
        c.start()

    gat_copies = [
        pltpu.make_async_copy(
            emb_hbm.at[g].at[idx_v.at[j]],
            gat_v.at[j * DIM + g], sem)
        for j in range(NCHUNK)
        for g in range(DIM)
    ]
    for c in gat_copies:
        c.start()
    for c in sgn_copies:
        c.wait()

    def tanh_chunk(i, _):
        x = sgn_v[pl.ds(i * 16, 16)]
        e = jnp.exp(-2.0 * jnp.abs(x))
        sgn_v[pl.ds(i * 16, 16)] = jnp.sign(x) * (1.0 - e) / (1.0 + e)
        return 0

    lax.fori_loop(0, NBLK, tanh_chunk, 0)
    for c in gat_copies:
        c.wait()

    def blk(b, accs):
        w_vec = sgn_v[pl.ds(b * 16, 16)]
        j = b // (CHUNK // 16)
        q = b % (CHUNK // 16)
        new = []
        for g in range(DIM):
            vals = gat_v[j * DIM + g, pl.ds(q * 16, 16)]
            new.append(accs[g] + w_vec * vals)
        return tuple(new)

    z = jnp.zeros((16,), jnp.float32)
    accs = lax.fori_loop(0, NBLK, blk, (z,) * DIM)

    for g in range(DIM):
        tmp_v[pl.ds(g * 16, 16)] = accs[g]
    dvec = lax.iota(jnp.int32, 16) * 16
    for h in range(2):
        s = jnp.zeros((16,), jnp.float32)
        for l in range(16):
            s = s + plsc.load_gather(tmp_v, [dvec + (h * 256 + l)])
        part_v[pl.ds(h * 16, 16)] = s
    pltpu.sync_copy(part_v, out_hbm.at[wid])


def _reduce_body(p_ref, o_ref):
    o_ref[...] = jnp.sum(p_ref[...], axis=0, keepdims=True)


@jax.jit
def kernel(sequence, hash_embedding, sign_weight):
    seq = sequence.astype(jnp.int32)
    sc = pl.kernel(
        _sc_body,
        out_type=jax.ShapeDtypeStruct((NW, DIM), jnp.float32),
        mesh=plsc.VectorSubcoreMesh(core_axis_name="c", subcore_axis_name="s"),
        scratch_types=[
            pltpu.VMEM((NCHUNK, CHUNK), jnp.int32),
            pltpu.VMEM((NCHUNK * DIM, CHUNK), jnp.float32),
            pltpu.VMEM((TPW,), jnp.float32),
            pltpu.VMEM((TPW,), jnp.float32),
            pltpu.VMEM((DIM,), jnp.float32),
            pltpu.SemaphoreType.DMA,
            pltpu.SemaphoreType.DMA,
        ],
        compiler_params=pltpu.CompilerParams(
            use_tc_tiling_on_sc=False, needs_layout_passes=False
        ),
    )
    partials = sc(seq, hash_embedding.T, sign_weight)
    out = pl.pallas_call(
        _reduce_body,
        out_shape=jax.ShapeDtypeStruct((1, DIM), jnp.float32),
    )(partials)
    return out.reshape(DIM)


# speedup vs baseline: 8.1237x; 8.1237x over previous
import functools

import jax
import jax.numpy as jnp
from jax import lax
from jax.experimental import pallas as pl
from jax.experimental.pallas import tpu as pltpu
from jax.experimental.pallas import tpu_sc as plsc

SEQ = 16384
DIM = 32
NC = 2
NS = 16
NW = NC * NS
TPW = SEQ // NW
CHUNK = 128
NCHUNK = TPW // CHUNK
NBLK = TPW // 16


def _sc_body(seq_hbm, emb_hbm, sgn_hbm, out_hbm,
             idx_v, rows_v, sgn_v, part_v, sem, ssem):
    wid = lax.axis_index("s") * NC + lax.axis_index("c")
    base = wid * TPW

    for j in range(NCHUNK):
        pltpu.sync_copy(seq_hbm.at[pl.ds(base + j * CHUNK, CHUNK)], idx_v.at[j])

    sgn_copies = [
        pltpu.make_async_copy(sgn_hbm.at[idx_v.at[j]],
                              sgn_v.at[pl.ds(j * CHUNK, CHUNK)], ssem)
        for j in range(NCHUNK)
    ]
    for c in sgn_copies:
        c.start()

    def fire_block(i):
        j = i // (CHUNK // 16)
        q = i % (CHUNK // 16)
        c_vec = idx_v[j, pl.ds(q * 16, 16)]
        t0 = i * 16
        for k in range(16):
            pltpu.make_async_copy(emb_hbm.at[c_vec[k]], rows_v.at[t0 + k], sem).start()

    def drain_block():
        for _ in range(16):
            pltpu.make_async_copy(emb_hbm.at[0], rows_v.at[0], sem).wait()

    def pipe(i, _):
        fire_block(i)
        drain_block()
        return 0

    fire_block(0)
    lax.fori_loop(1, NBLK, pipe, 0)
    drain_block()

    for c in sgn_copies:
        c.wait()

    def tanh_chunk(i, _):
        x = sgn_v[pl.ds(i * 16, 16)]
        e = jnp.exp(-2.0 * jnp.abs(x))
        sgn_v[pl.ds(i * 16, 16)] = jnp.sign(x) * (1.0 - e) / (1.0 + e)
        return 0

    lax.fori_loop(0, NBLK, tanh_chunk, 0)

    def blk(i, carry):
        a0, a1 = carry
        s_vec = sgn_v[pl.ds(i * 16, 16)]
        t0 = i * 16
        for k in range(16):
            s = s_vec[k]
            a0 = a0 + s * rows_v[t0 + k, pl.ds(0, 16)]
            a1 = a1 + s * rows_v[t0 + k, pl.ds(16, 16)]
        return (a0, a1)

    z = jnp.zeros((16,), jnp.float32)
    a0, a1 = lax.fori_loop(0, NBLK, blk, (z, z))
    part_v[pl.ds(0, 16)] = a0
    part_v[pl.ds(16, 16)] = a1
    pltpu.sync_copy(part_v, out_hbm.at[wid])


def _reduce_body(p_ref, o_ref):
    o_ref[...] = jnp.sum(p_ref[...], axis=0, keepdims=True)


@jax.jit
def kernel(sequence, hash_embedding, sign_weight):
    seq = sequence.astype(jnp.int32)
    sc = pl.kernel(
        _sc_body,
        out_type=jax.ShapeDtypeStruct((NW, DIM), jnp.float32),
        mesh=plsc.VectorSubcoreMesh(core_axis_name="c", subcore_axis_name="s"),
        scratch_types=[
            pltpu.VMEM((NCHUNK, CHUNK), jnp.int32),
            pltpu.VMEM((TPW, DIM), jnp.float32),
            pltpu.VMEM((TPW,), jnp.float32),
            pltpu.VMEM((DIM,), jnp.float32),
            pltpu.SemaphoreType.DMA,
            pltpu.SemaphoreType.DMA,
        ],
    )
    partials = sc(seq, hash_embedding, sign_weight)
    out = pl.pallas_call(
        _reduce_body,
        out_shape=jax.ShapeDtypeStruct((1, DIM), jnp.float32),
    )(partials)
    return out.reshape(DIM)


# speedup vs baseline: 15.6035x; 1.9207x over previous
import functools

import jax
import jax.numpy as jnp
from jax import lax
from jax.experimental import pallas as pl
from jax.experimental.pallas import tpu as pltpu
from jax.experimental.pallas import tpu_sc as plsc

SEQ = 16384
DIM = 32
NC = 2
NS = 16
NW = NC * NS
TPW = SEQ // NW
CHUNK = 128
NCHUNK = TPW // CHUNK
NGRP = TPW // 16
SUB = 8


def _sc_body(seq_hbm, emb_hbm, sgn_hbm, out_hbm,
             idx_v, tile_v, sgn_v, part_v, sem, ssem):
    wid = lax.axis_index("s") * NC + lax.axis_index("c")
    base = wid * TPW

    for j in range(NCHUNK):
        pltpu.sync_copy(seq_hbm.at[pl.ds(base + j * CHUNK, CHUNK)], idx_v.at[j])

    sgn_copies = [
        pltpu.make_async_copy(sgn_hbm.at[idx_v.at[j]],
                              sgn_v.at[pl.ds(j * CHUNK, CHUNK)], ssem)
        for j in range(NCHUNK)
    ]
    for c in sgn_copies:
        c.start()
    for c in sgn_copies:
        c.wait()

    def tanh_chunk(i, _):
        x = sgn_v[pl.ds(i * 16, 16)]
        e = jnp.exp(-2.0 * jnp.abs(x))
        sgn_v[pl.ds(i * 16, 16)] = jnp.sign(x) * (1.0 - e) / (1.0 + e)
        return 0

    lax.fori_loop(0, NGRP, tanh_chunk, 0)

    emb3 = emb_hbm.reshape(4, 8, emb_hbm.shape[-1])
    iota = lax.iota(jnp.int32, 16)
    dt_lo = iota // 8
    dt_hi = dt_lo + 2
    ds_sel = iota % 8

    def fire_sub(c_vec, s):
        for k in range(SUB):
            c = c_vec[s * SUB + k]
            start = pl.multiple_of(lax.shift_right_logical(c, 7) * 128, 128)
            for dt in range(4):
                pltpu.make_async_copy(
                    emb3.at[dt, :, pl.ds(start, 128)],
                    tile_v.at[k, dt], sem).start()

    def drain_sub():
        for _ in range(SUB * 4):
            pltpu.make_async_copy(
                emb3.at[0, :, pl.ds(0, 128)], tile_v.at[0, 0], sem).wait()

    def extract_sub(c_vec, w_vec, s, a_lo, a_hi):
        for k in range(SUB):
            rl = jnp.full((16,), c_vec[s * SUB + k] & 127, jnp.int32)
            tok = jnp.full((16,), k, jnp.int32)
            v_lo = plsc.load_gather(tile_v, [tok, dt_lo, ds_sel, rl])
            v_hi = plsc.load_gather(tile_v, [tok, dt_hi, ds_sel, rl])
            w = w_vec[s * SUB + k]
            a_lo = a_lo + w * v_lo
            a_hi = a_hi + w * v_hi
        return a_lo, a_hi

    def grp(i, carry):
        a_lo, a_hi = carry
        j = i // (CHUNK // 16)
        q = i % (CHUNK // 16)
        c_vec = idx_v[j, pl.ds(q * 16, 16)]
        w_vec = sgn_v[pl.ds(i * 16, 16)]
        for s in range(2):
            fire_sub(c_vec, s)
            drain_sub()
            a_lo, a_hi = extract_sub(c_vec, w_vec, s, a_lo, a_hi)
        return (a_lo, a_hi)

    z = jnp.zeros((16,), jnp.float32)
    a_lo, a_hi = lax.fori_loop(0, NGRP, grp, (z, z))
    part_v[pl.ds(0, 16)] = a_lo
    part_v[pl.ds(16, 16)] = a_hi
    pltpu.sync_copy(part_v, out_hbm.at[wid])


def _reduce_body(p_ref, o_ref):
    o_ref[...] = jnp.sum(p_ref[...], axis=0, keepdims=True)


@jax.jit
def kernel(sequence, hash_embedding, sign_weight):
    seq = sequence.astype(jnp.int32)
    sc = pl.kernel(
        _sc_body,
        out_type=jax.ShapeDtypeStruct((NW, DIM), jnp.float32),
        mesh=plsc.VectorSubcoreMesh(core_axis_name="c", subcore_axis_name="s"),
        scratch_types=[
            pltpu.VMEM((NCHUNK, CHUNK), jnp.int32),
            pltpu.VMEM((SUB, 4, 8, 128), jnp.float32),
            pltpu.VMEM((TPW,), jnp.float32),
            pltpu.VMEM((DIM,), jnp.float32),
            pltpu.SemaphoreType.DMA,
            pltpu.SemaphoreType.DMA,
        ],
        compiler_params=pltpu.CompilerParams(needs_layout_passes=False),
    )
    partials = sc(seq, hash_embedding.T, sign_weight)
    out = pl.pallas_call(
        _reduce_body,
        out_shape=jax.ShapeDtypeStruct((1, DIM), jnp.float32),
    )(partials)
    return out.reshape(DIM)


# speedup vs baseline: 18.7505x; 1.2017x over previous
import functools

import jax
import jax.numpy as jnp
from jax import lax
from jax.experimental import pallas as pl
from jax.experimental.pallas import tpu as pltpu
from jax.experimental.pallas import tpu_sc as plsc

SEQ = 16384
DIM = 32
NC = 2
NS = 16
NW = NC * NS
TPW = SEQ // NW
CHUNK = 128
NCHUNK = TPW // CHUNK
NGRP = TPW // 16
SUB = 8


def _sc_body(seq_hbm, emb_hbm, sgn_hbm, out_hbm,
             idx_v, tile_v, sgn_v, part_v, sem_a, sem_b, ssem):
    wid = lax.axis_index("s") * NC + lax.axis_index("c")
    base = wid * TPW

    for j in range(NCHUNK):
        pltpu.sync_copy(seq_hbm.at[pl.ds(base + j * CHUNK, CHUNK)], idx_v.at[j])

    sgn_copies = [
        pltpu.make_async_copy(sgn_hbm.at[idx_v.at[j]],
                              sgn_v.at[pl.ds(j * CHUNK, CHUNK)], ssem)
        for j in range(NCHUNK)
    ]
    for c in sgn_copies:
        c.start()
    for c in sgn_copies:
        c.wait()

    def tanh_chunk(i, _):
        x = sgn_v[pl.ds(i * 16, 16)]
        e = jnp.exp(-2.0 * jnp.abs(x))
        sgn_v[pl.ds(i * 16, 16)] = jnp.sign(x) * (1.0 - e) / (1.0 + e)
        return 0

    lax.fori_loop(0, NGRP, tanh_chunk, 0)

    emb3 = emb_hbm.reshape(4, 8, emb_hbm.shape[-1])
    iota = lax.iota(jnp.int32, 16)
    dt_lo = iota // 8
    dt_hi = dt_lo + 2
    ds_sel = iota % 8

    def load_cvec(g):
        j = g // (CHUNK // 16)
        q = g % (CHUNK // 16)
        return idx_v[j, pl.ds(q * 16, 16)]

    def fire_sub(g, s, buf, bsem):
        c_vec = load_cvec(g)
        for k in range(SUB):
            c = c_vec[s * SUB + k]
            start = pl.multiple_of(lax.shift_right_logical(c, 7) * 128, 128)
            for dt in range(4):
                pltpu.make_async_copy(
                    emb3.at[dt, :, pl.ds(start, 128)],
                    tile_v.at[buf, k, dt], bsem).start()

    def drain_sub(buf, bsem):
        for _ in range(SUB * 4):
            pltpu.make_async_copy(
                emb3.at[0, :, pl.ds(0, 128)], tile_v.at[buf, 0, 0], bsem).wait()

    def extract_sub(g, s, buf, a_lo, a_hi):
        c_vec = load_cvec(g)
        w_vec = sgn_v[pl.ds(g * 16, 16)]
        bvec = jnp.full((16,), buf, jnp.int32)
        for k in range(SUB):
            rl = jnp.full((16,), c_vec[s * SUB + k] & 127, jnp.int32)
            tok = jnp.full((16,), k, jnp.int32)
            v_lo = plsc.load_gather(tile_v, [bvec, tok, dt_lo, ds_sel, rl])
            v_hi = plsc.load_gather(tile_v, [bvec, tok, dt_hi, ds_sel, rl])
            w = w_vec[s * SUB + k]
            a_lo = a_lo + w * v_lo
            a_hi = a_hi + w * v_hi
        return a_lo, a_hi

    def pair(i, carry):
        a_lo, a_hi = carry
        fire_sub(i, 1, 1, sem_b)
        drain_sub(0, sem_a)
        a_lo, a_hi = extract_sub(i, 0, 0, a_lo, a_hi)

        @pl.when(i + 1 < NGRP)
        def _():
            fire_sub(i + 1, 0, 0, sem_a)

        drain_sub(1, sem_b)
        a_lo, a_hi = extract_sub(i, 1, 1, a_lo, a_hi)
        return (a_lo, a_hi)

    z = jnp.zeros((16,), jnp.float32)
    fire_sub(0, 0, 0, sem_a)
    a_lo, a_hi = lax.fori_loop(0, NGRP, pair, (z, z))
    part_v[pl.ds(0, 16)] = a_lo
    part_v[pl.ds(16, 16)] = a_hi
    pltpu.sync_copy(part_v, out_hbm.at[wid])


def _reduce_body(p_ref, o_ref):
    o_ref[...] = jnp.sum(p_ref[...], axis=0, keepdims=True)


@jax.jit
def kernel(sequence, hash_embedding, sign_weight):
    seq = sequence.astype(jnp.int32)
    sc = pl.kernel(
        _sc_body,
        out_type=jax.ShapeDtypeStruct((NW, DIM), jnp.float32),
        mesh=plsc.VectorSubcoreMesh(core_axis_name="c", subcore_axis_name="s"),
        scratch_types=[
            pltpu.VMEM((NCHUNK, CHUNK), jnp.int32),
            pltpu.VMEM((2, SUB, 4, 8, 128), jnp.float32),
            pltpu.VMEM((TPW,), jnp.float32),
            pltpu.VMEM((DIM,), jnp.float32),
            pltpu.SemaphoreType.DMA,
            pltpu.SemaphoreType.DMA,
            pltpu.SemaphoreType.DMA,
        ],
        compiler_params=pltpu.CompilerParams(needs_layout_passes=False),
    )
    partials = sc(seq, hash_embedding.T, sign_weight)
    out = pl.pallas_call(
        _reduce_body,
        out_shape=jax.ShapeDtypeStruct((1, DIM), jnp.float32),
    )(partials)
    return out.reshape(DIM)
